# baseline (speedup 1.0000x reference)
import functools

import jax
import jax.numpy as jnp
from jax.experimental import pallas as pl
from jax.experimental.pallas import tpu as pltpu

E = 16
TOPK = 2
BLOCK_T = 512


def _moe_body(x_ref, gw_ref, ew_ref, o_ref):
    x = x_ref[...]
    logits = jax.lax.dot_general(
        x, gw_ref[...], (((1,), (1,)), ((), ())),
        preferred_element_type=jnp.float32)
    m = jnp.max(logits, axis=1, keepdims=True)
    p = jnp.exp(logits - m)
    probs = p / jnp.sum(p, axis=1, keepdims=True)

    iota = jax.lax.broadcasted_iota(jnp.int32, probs.shape, 1)
    v0 = jnp.max(probs, axis=1, keepdims=True)
    e0 = jnp.min(jnp.where(probs == v0, iota, E), axis=1, keepdims=True)
    mask0 = iota == e0
    rest = jnp.where(mask0, -jnp.inf, probs)
    v1 = jnp.max(rest, axis=1, keepdims=True)
    e1 = jnp.min(jnp.where(rest == v1, iota, E), axis=1, keepdims=True)
    mask1 = iota == e1

    inv = 1.0 / (v0 + v1)
    w = jnp.where(mask0, v0 * inv, 0.0) + jnp.where(mask1, v1 * inv, 0.0)

    scale = jax.lax.dot_general(
        w, ew_ref[...], (((1,), (0,)), ((), ())),
        preferred_element_type=jnp.float32)
    o_ref[...] = x * scale


@functools.partial(jax.jit, static_argnames=())
def kernel(norm_data, gate_w, expert_w):
    T, D = norm_data.shape
    grid = (T // BLOCK_T,)
    return pl.pallas_call(
        _moe_body,
        grid=grid,
        in_specs=[
            pl.BlockSpec((BLOCK_T, D), lambda i: (i, 0)),
            pl.BlockSpec((E, D), lambda i: (0, 0)),
            pl.BlockSpec((E, D), lambda i: (0, 0)),
        ],
        out_specs=pl.BlockSpec((BLOCK_T, D), lambda i: (i, 0)),
        out_shape=jax.ShapeDtypeStruct((T, D), norm_data.dtype),
        compiler_params=pltpu.CompilerParams(
            dimension_semantics=("arbitrary",),
        ),
    )(norm_data, gate_w, expert_w)


# speedup vs baseline: 1.1506x; 1.1506x over previous
import functools

import jax
import jax.numpy as jnp
from jax.experimental import pallas as pl
from jax.experimental.pallas import tpu as pltpu

E = 16
TOPK = 2
BLOCK_T = 1024


def _moe_body(x_ref, gw_ref, ew_ref, o_ref):
    x = x_ref[...]
    logits = jax.lax.dot_general(
        x.astype(jnp.bfloat16), gw_ref[...].astype(jnp.bfloat16),
        (((1,), (1,)), ((), ())),
        preferred_element_type=jnp.float32)

    iota = jax.lax.broadcasted_iota(jnp.int32, logits.shape, 1)
    v0 = jnp.max(logits, axis=1, keepdims=True)
    e0 = jnp.min(jnp.where(logits == v0, iota, E), axis=1, keepdims=True)
    mask0 = iota == e0
    rest = jnp.where(mask0, -jnp.inf, logits)
    v1 = jnp.max(rest, axis=1, keepdims=True)
    e1 = jnp.min(jnp.where(rest == v1, iota, E), axis=1, keepdims=True)
    mask1 = iota == e1

    w0 = 1.0 / (1.0 + jnp.exp(v1 - v0))
    w = jnp.where(mask0, w0, 0.0) + jnp.where(mask1, 1.0 - w0, 0.0)

    scale = jax.lax.dot_general(
        w, ew_ref[...], (((1,), (0,)), ((), ())),
        preferred_element_type=jnp.float32)
    o_ref[...] = x * scale


@functools.partial(jax.jit, static_argnames=())
def kernel(norm_data, gate_w, expert_w):
    T, D = norm_data.shape
    grid = (T // BLOCK_T,)
    return pl.pallas_call(
        _moe_body,
        grid=grid,
        in_specs=[
            pl.BlockSpec((BLOCK_T, D), lambda i: (i, 0)),
            pl.BlockSpec((E, D), lambda i: (0, 0)),
            pl.BlockSpec((E, D), lambda i: (0, 0)),
        ],
        out_specs=pl.BlockSpec((BLOCK_T, D), lambda i: (i, 0)),
        out_shape=jax.ShapeDtypeStruct((T, D), norm_data.dtype),
        compiler_params=pltpu.CompilerParams(
            dimension_semantics=("arbitrary",),
        ),
    )(norm_data, gate_w, expert_w)
